# baseline (speedup 1.0000x reference)
import functools

import jax
import jax.numpy as jnp
from jax import lax
from jax.experimental import pallas as pl
from jax.experimental.pallas import tpu as pltpu
from jax.experimental.pallas import tpu_sc as plsc

_B = 16384
_D = 64
_NC = 2
_NS = 16
_NW = _NC * _NS
_BW = _B // _NW
_L = 16


def _make_kernel():
    mesh = plsc.VectorSubcoreMesh(core_axis_name="c", subcore_axis_name="s")

    @functools.partial(
        pl.kernel,
        mesh=mesh,
        compiler_params=pltpu.CompilerParams(
            needs_layout_passes=False, use_tc_tiling_on_sc=False),
        out_type=[
            jax.ShapeDtypeStruct((_B,), jnp.float32),
            jax.ShapeDtypeStruct((_B,), jnp.float32),
        ],
        scratch_types=[
            pltpu.VMEM((_BW,), jnp.int32),
            pltpu.VMEM((_BW,), jnp.int32),
            pltpu.VMEM((_BW,), jnp.int32),
            pltpu.VMEM((_BW, _D), jnp.float32),
            pltpu.VMEM((_BW, _D), jnp.float32),
            pltpu.VMEM((_BW, _D), jnp.float32),
            pltpu.VMEM((_BW,), jnp.float32),
            pltpu.SemaphoreType.DMA,
            pltpu.SemaphoreType.DMA,
            pltpu.SemaphoreType.DMA,
        ],
    )
    def trans_e(ph, pr, pt, nh, nr, nt, ent, rel, pos_out, neg_out,
                idx_h, idx_r, idx_t, hrows, rrows, trows, obuf,
                sem_h, sem_r, sem_t):
        wid = lax.axis_index("s") * _NC + lax.axis_index("c")
        base = wid * _BW
        lanes = lax.iota(jnp.int32, _L)

        def one_side(h_hbm, r_hbm, t_hbm, out_hbm):
            pltpu.sync_copy(h_hbm.at[pl.ds(base, _BW)], idx_h)
            pltpu.sync_copy(r_hbm.at[pl.ds(base, _BW)], idx_r)
            pltpu.sync_copy(t_hbm.at[pl.ds(base, _BW)], idx_t)
            ch = pltpu.async_copy(ent.at[idx_h], hrows, sem_h)
            cr = pltpu.async_copy(rel.at[idx_r], rrows, sem_r)
            ct = pltpu.async_copy(ent.at[idx_t], trows, sem_t)
            ch.wait()
            cr.wait()
            ct.wait()

            def group(g, carry):
                vec = jnp.zeros((_L,), jnp.float32)
                for j in range(_L):
                    i = g * _L + j
                    acc = jnp.zeros((_L,), jnp.float32)
                    for c in range(_D // _L):
                        hv = hrows[i, pl.ds(c * _L, _L)]
                        rv = rrows[i, pl.ds(c * _L, _L)]
                        tv = trows[i, pl.ds(c * _L, _L)]
                        acc = acc + jnp.abs(hv + rv - tv)
                    vec = jnp.where(lanes == j, jnp.sum(acc), vec)
                obuf[pl.ds(g * _L, _L)] = vec
                return carry

            lax.fori_loop(0, _BW // _L, group, 0)
            pltpu.sync_copy(obuf, out_hbm.at[pl.ds(base, _BW)])

        one_side(ph, pr, pt, pos_out)
        one_side(nh, nr, nt, neg_out)

    return trans_e


_KERNEL = _make_kernel()


@jax.jit
def kernel(pos_triples, neg_triples, ent_embs, rel_embs):
    pos = pos_triples.astype(jnp.int32)
    neg = neg_triples.astype(jnp.int32)
    ph, pr, pt = pos[:, 0], pos[:, 1], pos[:, 2]
    nh, nr, nt = neg[:, 0], neg[:, 1], neg[:, 2]
    pos_dist, neg_dist = _KERNEL(ph, pr, pt, nh, nr, nt, ent_embs, rel_embs)
    return pos_dist, neg_dist
